# baseline (speedup 1.0000x reference)
import functools

import jax
import jax.numpy as jnp
from jax import lax
from jax.experimental import pallas as pl
from jax.experimental.pallas import tpu as pltpu
from jax.experimental.pallas import tpu_sc as plsc

N = 8192
BX = 4096
BY = 4096
NW = 32
R = 4
RPW = BX // NW
NCH = RPW // R
LANES = 16

_mesh = plsc.VectorSubcoreMesh(core_axis_name="c", subcore_axis_name="s")


@functools.partial(
    pl.kernel,
    out_type=jax.ShapeDtypeStruct((BX // R, R * BY), jnp.float32),
    mesh=_mesh,
    compiler_params=pltpu.CompilerParams(
        use_tc_tiling_on_sc=False, needs_layout_passes=False
    ),
    scratch_types=[
        pltpu.VMEM((NCH, 2 * R), jnp.int32),
        pltpu.VMEM((BY,), jnp.int32),
        pltpu.VMEM((LANES,), jnp.float32),
        pltpu.VMEM((2, 2 * R, N // 2), jnp.float32),
        pltpu.VMEM((2, R * BY), jnp.float32),
        pltpu.SemaphoreType.DMA,
        pltpu.SemaphoreType.DMA,
        pltpu.SemaphoreType.DMA,
        pltpu.SemaphoreType.DMA,
    ],
)
def _sc_gather_exp(tab, x2, y, ninv, out,
                   x2_v, y_v, ninv_v, rows_v, out_v, sg0, sg1, so0, so1):
    wid = lax.axis_index("s") * 2 + lax.axis_index("c")
    g0 = wid * NCH

    pltpu.sync_copy(x2.at[pl.ds(g0, NCH)], x2_v)
    pltpu.sync_copy(y, y_v)
    pltpu.sync_copy(ninv, ninv_v)
    ninv16 = ninv_v[...]

    sg = (sg0, sg1)
    so = (so0, so1)

    def start_gather(c):
        b = c & 1
        return pltpu.async_copy(tab.at[x2_v.at[c]], rows_v.at[b], sg[b])

    def compute(c):
        b = c & 1
        rows_b = rows_v.at[b]
        out_b = out_v.at[b]

        def blk(j, carry):
            r = j >> 8
            jb = j & 255
            yv = y_v[pl.ds(jb * LANES, LANES)]
            yh = (yv >> 12) + 2 * r
            yl = yv & 4095
            v = plsc.load_gather(rows_b, [yh, yl])
            out_b[pl.ds(r * BY + jb * LANES, LANES)] = jnp.exp(v * ninv16)
            return carry

        lax.fori_loop(0, R * (BY // LANES), blk, 0)

    gh = [start_gather(0), start_gather(1)]
    sh = [None, None]
    for c in range(NCH):
        b = c & 1
        gh[b].wait()
        if sh[b] is not None:
            sh[b].wait()
        compute(c)
        sh[b] = pltpu.async_copy(out_v.at[b], out.at[g0 + c], so[b])
        if c + 2 < NCH:
            gh[b] = start_gather(c + 2)
    sh[0].wait()
    sh[1].wait()


def kernel(x, y, distance, scale):
    s = jnp.clip(jax.nn.softplus(scale), 1e-10, 10000.0)
    ninv = jnp.broadcast_to(-1.0 / s, (LANES,)).astype(jnp.float32)
    xi = x.astype(jnp.int32)
    x2 = jnp.stack([xi * 2, xi * 2 + 1], axis=1).reshape(BX // R, 2 * R)
    yi = y.astype(jnp.int32)
    tab = distance.reshape(2 * N, N // 2)
    out = _sc_gather_exp(tab, x2, yi, ninv)
    return out.reshape(BX, BY)
